# baseline (speedup 1.0000x reference)
import functools

import jax
import jax.numpy as jnp
from jax.experimental import pallas as pl
import jax.experimental.pallas.tpu as pltpu

B, S, D, E = 4, 2048, 4096, 64
S_CHUNK = 128
N_SCHUNKS = S // S_CHUNK
D_CHUNK = 512
N_DCHUNKS = D // D_CHUNK


def _mean_ln_kernel(state_ref, scale_ref, bias_ref, h_ref, acc_ref):
    i = pl.program_id(0)

    @pl.when(i == 0)
    def _init():
        acc_ref[...] = jnp.zeros_like(acc_ref)

    acc_ref[...] += jnp.sum(state_ref[...], axis=1)

    @pl.when(i == N_SCHUNKS - 1)
    def _finish():
        m = acc_ref[...] * (1.0 / S)
        mu = jnp.mean(m, axis=-1, keepdims=True)
        var = jnp.mean((m - mu) ** 2, axis=-1, keepdims=True)
        h = (m - mu) * jax.lax.rsqrt(var + 1e-5)
        h_ref[...] = h * scale_ref[...] + bias_ref[...]


def _mlp1_kernel(h_ref, w1_ref, b1_ref, g_ref):
    z = jnp.dot(h_ref[...], w1_ref[...], preferred_element_type=jnp.float32)
    z = z + b1_ref[...]
    g_ref[...] = z * 0.5 * (1.0 + jax.lax.erf(z * 0.7071067811865476))


def _mlp2_att_kernel(g_ref, w2_ref, b2_ref, emb_ref, cb_ref, probs_ref, acc_ref):
    j = pl.program_id(0)

    @pl.when(j == 0)
    def _init():
        acc_ref[...] = jnp.zeros_like(acc_ref)

    p = jnp.dot(g_ref[...], w2_ref[...], preferred_element_type=jnp.float32)
    p = p + b2_ref[...]
    acc_ref[...] += jnp.dot(p, emb_ref[...].T, preferred_element_type=jnp.float32)

    @pl.when(j == N_DCHUNKS - 1)
    def _finish():
        att = acc_ref[...] + cb_ref[...]
        att = att - jnp.max(att, axis=-1, keepdims=True)
        ex = jnp.exp(att)
        probs_ref[...] = ex / jnp.sum(ex, axis=-1, keepdims=True)


def kernel(state, ln_scale, ln_bias, W1, b1, W2, b2, expert_emb, centrality, spatial, current_expert_idx):
    scale2 = ln_scale.reshape(1, D)
    bias2 = ln_bias.reshape(1, D)
    b1_2 = b1.reshape(1, D)
    b2_2 = b2.reshape(1, D)
    spatial_row = jax.lax.dynamic_index_in_dim(spatial, current_expert_idx, 0, keepdims=False)
    combined_bias = (centrality + spatial_row).reshape(1, E)

    h = pl.pallas_call(
        _mean_ln_kernel,
        grid=(N_SCHUNKS,),
        in_specs=[
            pl.BlockSpec((B, S_CHUNK, D), lambda i: (0, i, 0)),
            pl.BlockSpec((1, D), lambda i: (0, 0)),
            pl.BlockSpec((1, D), lambda i: (0, 0)),
        ],
        out_specs=pl.BlockSpec((B, D), lambda i: (0, 0)),
        out_shape=jax.ShapeDtypeStruct((B, D), jnp.float32),
        scratch_shapes=[pltpu.VMEM((B, D), jnp.float32)],
    )(state, scale2, bias2)

    g = pl.pallas_call(
        _mlp1_kernel,
        grid=(N_DCHUNKS,),
        in_specs=[
            pl.BlockSpec((B, D), lambda j: (0, 0)),
            pl.BlockSpec((D, D_CHUNK), lambda j: (0, j)),
            pl.BlockSpec((1, D_CHUNK), lambda j: (0, j)),
        ],
        out_specs=pl.BlockSpec((B, D_CHUNK), lambda j: (0, j)),
        out_shape=jax.ShapeDtypeStruct((B, D), jnp.float32),
    )(h, W1, b1_2)

    probs = pl.pallas_call(
        _mlp2_att_kernel,
        grid=(N_DCHUNKS,),
        in_specs=[
            pl.BlockSpec((B, D), lambda j: (0, 0)),
            pl.BlockSpec((D, D_CHUNK), lambda j: (0, j)),
            pl.BlockSpec((1, D_CHUNK), lambda j: (0, j)),
            pl.BlockSpec((E, D_CHUNK), lambda j: (0, j)),
            pl.BlockSpec((1, E), lambda j: (0, 0)),
        ],
        out_specs=pl.BlockSpec((B, E), lambda j: (0, 0)),
        out_shape=jax.ShapeDtypeStruct((B, E), jnp.float32),
        scratch_shapes=[pltpu.VMEM((B, E), jnp.float32)],
    )(g, W2, b2_2, expert_emb, combined_bias)

    return probs


# speedup vs baseline: 1.0963x; 1.0963x over previous
import jax
import jax.numpy as jnp
from jax.experimental import pallas as pl
import jax.experimental.pallas.tpu as pltpu

B, S, D, E = 4, 2048, 4096, 64
S_CHUNK = 128
N_SCHUNKS = S // S_CHUNK
D_CHUNK = 512
N_DCHUNKS = D // D_CHUNK
PH1 = N_SCHUNKS
PH2 = N_SCHUNKS + N_DCHUNKS
NSTEPS = N_SCHUNKS + 2 * N_DCHUNKS


def _fused_kernel(state_ref, scale_ref, bias_ref, w1_ref, b1_ref,
                  w2_ref, b2_ref, emb_ref, cb_ref, probs_ref,
                  macc_ref, h_ref, g_ref, att_ref):
    i = pl.program_id(0)

    @pl.when(i == 0)
    def _init():
        macc_ref[...] = jnp.zeros_like(macc_ref)
        att_ref[...] = jnp.zeros_like(att_ref)

    @pl.when(i < PH1)
    def _mean_phase():
        macc_ref[...] += jnp.sum(state_ref[...], axis=1)

    @pl.when(i == PH1 - 1)
    def _layernorm():
        m = macc_ref[...] * (1.0 / S)
        mu = jnp.mean(m, axis=-1, keepdims=True)
        var = jnp.mean((m - mu) ** 2, axis=-1, keepdims=True)
        h = (m - mu) * jax.lax.rsqrt(var + 1e-5)
        h_ref[...] = h * scale_ref[...] + bias_ref[...]

    @pl.when((i >= PH1) & (i < PH2))
    def _mlp1_phase():
        z = jnp.dot(h_ref[...], w1_ref[...], preferred_element_type=jnp.float32)
        z = z + b1_ref[...]
        g = z * 0.5 * (1.0 + jax.lax.erf(z * 0.7071067811865476))
        g_ref[:, pl.ds((i - PH1) * D_CHUNK, D_CHUNK)] = g

    @pl.when(i >= PH2)
    def _mlp2_phase():
        p = jnp.dot(g_ref[...], w2_ref[...], preferred_element_type=jnp.float32)
        p = p + b2_ref[...]
        att_ref[...] += jnp.dot(p, emb_ref[...].T,
                                preferred_element_type=jnp.float32)

    @pl.when(i == NSTEPS - 1)
    def _finish():
        att = att_ref[...] + cb_ref[...]
        att = att - jnp.max(att, axis=-1, keepdims=True)
        ex = jnp.exp(att)
        probs_ref[...] = ex / jnp.sum(ex, axis=-1, keepdims=True)


def _clamp(lo, x, hi):
    return jnp.minimum(jnp.maximum(x, lo), hi)


def kernel(state, ln_scale, ln_bias, W1, b1, W2, b2, expert_emb, centrality, spatial, current_expert_idx):
    scale2 = ln_scale.reshape(1, D)
    bias2 = ln_bias.reshape(1, D)
    b1_2 = b1.reshape(1, D)
    b2_2 = b2.reshape(1, D)
    spatial_row = jax.lax.dynamic_index_in_dim(spatial, current_expert_idx, 0, keepdims=False)
    combined_bias = (centrality + spatial_row).reshape(1, E)

    probs = pl.pallas_call(
        _fused_kernel,
        grid=(NSTEPS,),
        in_specs=[
            pl.BlockSpec((B, S_CHUNK, D), lambda i: (0, jnp.minimum(i, N_SCHUNKS - 1), 0)),
            pl.BlockSpec((1, D), lambda i: (0, 0)),
            pl.BlockSpec((1, D), lambda i: (0, 0)),
            pl.BlockSpec((D, D_CHUNK), lambda i: (0, _clamp(0, i - PH1, N_DCHUNKS - 1))),
            pl.BlockSpec((1, D_CHUNK), lambda i: (0, _clamp(0, i - PH1, N_DCHUNKS - 1))),
            pl.BlockSpec((D, D_CHUNK), lambda i: (0, _clamp(0, i - PH2, N_DCHUNKS - 1))),
            pl.BlockSpec((1, D_CHUNK), lambda i: (0, _clamp(0, i - PH2, N_DCHUNKS - 1))),
            pl.BlockSpec((E, D_CHUNK), lambda i: (0, _clamp(0, i - PH2, N_DCHUNKS - 1))),
            pl.BlockSpec((1, E), lambda i: (0, 0)),
        ],
        out_specs=pl.BlockSpec((B, E), lambda i: (0, 0)),
        out_shape=jax.ShapeDtypeStruct((B, E), jnp.float32),
        scratch_shapes=[
            pltpu.VMEM((B, D), jnp.float32),
            pltpu.VMEM((B, D), jnp.float32),
            pltpu.VMEM((B, D), jnp.float32),
            pltpu.VMEM((B, E), jnp.float32),
        ],
    )(state, scale2, bias2, W1, b1_2, W2, b2_2, expert_emb, combined_bias)

    return probs
